# baseline (speedup 1.0000x reference)
import jax
import jax.numpy as jnp
from jax.experimental import pallas as pl
from jax.experimental.pallas import tpu as pltpu

_SPLITS = (576, 8640, 8640, 576)


def _passthrough(img_in, lab_in, img_out, lab_out, bufs, sem_in, sem_out, sem_lab):
    starts = [sum(_SPLITS[:i]) for i in range(len(_SPLITS))]
    cin = [
        pltpu.make_async_copy(
            img_in.at[pl.ds(starts[i], n)], bufs[i], sem_in.at[i]
        )
        for i, n in enumerate(_SPLITS)
    ]
    cout = [
        pltpu.make_async_copy(
            bufs[i], img_out.at[pl.ds(starts[i], n)], sem_out.at[i]
        )
        for i, n in enumerate(_SPLITS)
    ]
    clab = pltpu.make_async_copy(lab_in, lab_out, sem_lab)
    clab.start()
    cin[0].start()
    cin[1].start()
    for i in range(len(_SPLITS)):
        cin[i].wait()
        cout[i].start()
        if i + 2 < len(_SPLITS):
            cin[i + 2].start()
    for c in cout:
        c.wait()
    clab.wait()


def kernel(images, labels):
    B, H, W, C = images.shape
    img2 = images.transpose(0, 3, 1, 2).reshape(B * C * H, W)
    rows, cols = img2.shape
    out_img, out_lab = pl.pallas_call(
        _passthrough,
        in_specs=[
            pl.BlockSpec(memory_space=pl.ANY),
            pl.BlockSpec(memory_space=pl.ANY),
        ],
        out_specs=[
            pl.BlockSpec(memory_space=pl.ANY),
            pl.BlockSpec(memory_space=pl.ANY),
        ],
        out_shape=[
            jax.ShapeDtypeStruct(img2.shape, img2.dtype),
            jax.ShapeDtypeStruct(labels.shape, labels.dtype),
        ],
        scratch_shapes=[
            [pltpu.VMEM((n, cols), img2.dtype) for n in _SPLITS],
            pltpu.SemaphoreType.DMA((len(_SPLITS),)),
            pltpu.SemaphoreType.DMA((len(_SPLITS),)),
            pltpu.SemaphoreType.DMA,
        ],
    )(img2, labels)
    return (out_img.reshape(B, C, H, W).transpose(0, 2, 3, 1), out_lab)


# speedup vs baseline: 1.0234x; 1.0234x over previous
import jax
import jax.numpy as jnp
from jax.experimental import pallas as pl
from jax.experimental.pallas import tpu as pltpu

_SPLITS = (1728, 7488, 7488, 1728)


def _passthrough(img_in, lab_in, img_out, lab_out, bufs, sem_in, sem_out, sem_lab):
    starts = [sum(_SPLITS[:i]) for i in range(len(_SPLITS))]
    cin = [
        pltpu.make_async_copy(
            img_in.at[pl.ds(starts[i], n)], bufs[i], sem_in.at[i]
        )
        for i, n in enumerate(_SPLITS)
    ]
    cout = [
        pltpu.make_async_copy(
            bufs[i], img_out.at[pl.ds(starts[i], n)], sem_out.at[i]
        )
        for i, n in enumerate(_SPLITS)
    ]
    clab = pltpu.make_async_copy(lab_in, lab_out, sem_lab)
    clab.start()
    cin[0].start()
    cin[1].start()
    for i in range(len(_SPLITS)):
        cin[i].wait()
        cout[i].start()
        if i + 2 < len(_SPLITS):
            cin[i + 2].start()
    for c in cout:
        c.wait()
    clab.wait()


def kernel(images, labels):
    B, H, W, C = images.shape
    img2 = images.transpose(0, 3, 1, 2).reshape(B * C * H, W)
    rows, cols = img2.shape
    out_img, out_lab = pl.pallas_call(
        _passthrough,
        in_specs=[
            pl.BlockSpec(memory_space=pl.ANY),
            pl.BlockSpec(memory_space=pl.ANY),
        ],
        out_specs=[
            pl.BlockSpec(memory_space=pl.ANY),
            pl.BlockSpec(memory_space=pl.ANY),
        ],
        out_shape=[
            jax.ShapeDtypeStruct(img2.shape, img2.dtype),
            jax.ShapeDtypeStruct(labels.shape, labels.dtype),
        ],
        scratch_shapes=[
            [pltpu.VMEM((n, cols), img2.dtype) for n in _SPLITS],
            pltpu.SemaphoreType.DMA((len(_SPLITS),)),
            pltpu.SemaphoreType.DMA((len(_SPLITS),)),
            pltpu.SemaphoreType.DMA,
        ],
    )(img2, labels)
    return (out_img.reshape(B, C, H, W).transpose(0, 2, 3, 1), out_lab)


# speedup vs baseline: 1.0247x; 1.0012x over previous
import jax
import jax.numpy as jnp
from jax.experimental import pallas as pl
from jax.experimental.pallas import tpu as pltpu

_SPLITS = (2016, 7200, 7200, 2016)


def _passthrough(img_in, lab_in, img_out, lab_out, bufs, sem_in, sem_out, sem_lab):
    starts = [sum(_SPLITS[:i]) for i in range(len(_SPLITS))]
    cin = [
        pltpu.make_async_copy(
            img_in.at[pl.ds(starts[i], n)], bufs[i], sem_in.at[i]
        )
        for i, n in enumerate(_SPLITS)
    ]
    cout = [
        pltpu.make_async_copy(
            bufs[i], img_out.at[pl.ds(starts[i], n)], sem_out.at[i]
        )
        for i, n in enumerate(_SPLITS)
    ]
    clab = pltpu.make_async_copy(lab_in, lab_out, sem_lab)
    clab.start()
    cin[0].start()
    cin[1].start()
    for i in range(len(_SPLITS)):
        cin[i].wait()
        cout[i].start()
        if i + 2 < len(_SPLITS):
            cin[i + 2].start()
    for c in cout:
        c.wait()
    clab.wait()


def kernel(images, labels):
    B, H, W, C = images.shape
    img2 = images.transpose(0, 3, 1, 2).reshape(B * C * H, W)
    rows, cols = img2.shape
    out_img, out_lab = pl.pallas_call(
        _passthrough,
        in_specs=[
            pl.BlockSpec(memory_space=pl.ANY),
            pl.BlockSpec(memory_space=pl.ANY),
        ],
        out_specs=[
            pl.BlockSpec(memory_space=pl.ANY),
            pl.BlockSpec(memory_space=pl.ANY),
        ],
        out_shape=[
            jax.ShapeDtypeStruct(img2.shape, img2.dtype),
            jax.ShapeDtypeStruct(labels.shape, labels.dtype),
        ],
        scratch_shapes=[
            [pltpu.VMEM((n, cols), img2.dtype) for n in _SPLITS],
            pltpu.SemaphoreType.DMA((len(_SPLITS),)),
            pltpu.SemaphoreType.DMA((len(_SPLITS),)),
            pltpu.SemaphoreType.DMA,
        ],
    )(img2, labels)
    return (out_img.reshape(B, C, H, W).transpose(0, 2, 3, 1), out_lab)
